# Initial kernel scaffold; baseline (speedup 1.0000x reference)
import jax
import jax.numpy as jnp
from jax.experimental import pallas as pl


def kernel(fragments, alphas, ptclds):
    raise NotImplementedError("write your pallas kernel here")


# speedup vs baseline: 1.1620x; 1.1620x over previous
import functools

import jax
import jax.numpy as jnp
from jax import lax
from jax.experimental import pallas as pl
from jax.experimental.pallas import tpu as pltpu
from jax.experimental.pallas import tpu_sc as plsc

NC = 2
NS = 16
NW = NC * NS
LANES = 16

STEP = 256
SUB = 128


def _transpose_body(x_ref, o_ref):
    o_ref[...] = x_ref[...].T


def _transpose_table(ptclds):
    C, P = ptclds.shape
    PB = 2048
    grid = (P + PB - 1) // PB
    return pl.pallas_call(
        _transpose_body,
        grid=(grid,),
        in_specs=[pl.BlockSpec((C, PB), lambda i: (0, i))],
        out_specs=pl.BlockSpec((PB, C), lambda i: (i, 0)),
        out_shape=jax.ShapeDtypeStruct((P, C), jnp.float32),
    )(ptclds)


def _make_sc_kernel(N, K, HW, C, P):
    n_pix = N * HW
    pix_per_tile = n_pix // NW
    n_steps = pix_per_tile // STEP
    tiles_per_img = HW // pix_per_tile

    mesh = plsc.VectorSubcoreMesh(
        core_axis_name="c", subcore_axis_name="s", num_cores=NC, num_subcores=NS
    )

    @functools.partial(
        pl.kernel,
        out_type=jax.ShapeDtypeStruct((N * C, HW), jnp.float32),
        mesh=mesh,
        compiler_params=pltpu.CompilerParams(
            needs_layout_passes=False, use_tc_tiling_on_sc=False
        ),
        scratch_types=[
            pltpu.VMEM((K, STEP), jnp.int32),
            pltpu.VMEM((K, STEP), jnp.float32),
            pltpu.VMEM((K * STEP, C), jnp.float32),
            pltpu.VMEM((C, STEP), jnp.float32),
            pltpu.SemaphoreType.DMA,
        ],
    )
    def sc_kernel(frag_hbm, alpha_hbm, table_hbm, out_hbm,
                  idx_v, alpha_v, rows_v, out_stage, dma_sem):
        cid = lax.axis_index("c")
        sid = lax.axis_index("s")
        wid = sid * NC + cid
        n = wid // tiles_per_img
        col0 = (wid % tiles_per_img) * pix_per_tile

        iota16 = lax.iota(jnp.int32, LANES)

        def step(s, carry):
            col = col0 + s * STEP
            pltpu.sync_copy(frag_hbm.at[n, :, pl.ds(col, STEP)], idx_v)
            pltpu.sync_copy(alpha_hbm.at[n, :, pl.ds(col, STEP)], alpha_v)

            copies = []
            for k in range(K):
                for hf in range(STEP // SUB):
                    cp = pltpu.async_copy(
                        table_hbm.at[idx_v.at[k, pl.ds(hf * SUB, SUB)]],
                        rows_v.at[pl.ds(k * STEP + hf * SUB, SUB), :],
                        dma_sem,
                    )
                    copies.append(cp)
            for cp in copies:
                cp.wait()

            def group(g, c2):
                gsl = pl.ds(g * LANES, LANES)
                a = [alpha_v[k, gsl] for k in range(K)]
                d = a[0]
                for k in range(1, K):
                    d = d + a[k]
                r = 1.0 / jnp.maximum(d, 1e-10)
                w = [ak * r for ak in a]
                pvec = g * LANES + iota16
                rowvecs = [pvec + k * STEP for k in range(K)]
                cvecs = [jnp.full((LANES,), c, jnp.int32) for c in range(C)]
                for c in range(C):
                    acc = w[0] * plsc.load_gather(rows_v, [rowvecs[0], cvecs[c]])
                    for k in range(1, K):
                        acc = acc + w[k] * plsc.load_gather(
                            rows_v, [rowvecs[k], cvecs[c]])
                    out_stage[c, gsl] = acc
                return c2

            lax.fori_loop(0, STEP // LANES, group, 0)

            pltpu.sync_copy(
                out_stage,
                out_hbm.at[pl.ds(n * C, C), pl.ds(col, STEP)],
            )
            return carry

        lax.fori_loop(0, n_steps, step, 0)

    return sc_kernel


def kernel(fragments, alphas, ptclds):
    N, K, H, W = fragments.shape
    C, P = ptclds.shape
    HW = H * W

    table = _transpose_table(ptclds)
    frag = fragments.reshape(N, K, HW).astype(jnp.int32)
    alph = alphas.reshape(N, K, HW)

    sc_kernel = _make_sc_kernel(N, K, HW, C, P)
    out = sc_kernel(frag, alph, table)
    return out.reshape(N, C, H, W)
